# Initial kernel scaffold; baseline (speedup 1.0000x reference)
import jax
import jax.numpy as jnp
from jax.experimental import pallas as pl


def kernel(x, edge_index, batch, W_nbr, W_root, b):
    raise NotImplementedError("write your pallas kernel here")


# speedup vs baseline: 5.6204x; 5.6204x over previous
import functools

import jax
import jax.numpy as jnp
from jax import lax
from jax.experimental import pallas as pl
from jax.experimental.pallas import tpu as pltpu
from jax.experimental.pallas import tpu_sc as plsc

N = 10000
E = 320000
D = 128
G = 256

NC = 2
NS = 16
EPT = E // (NC * NS)
CH = 80
NCHUNK = EPT // CH
RPT = 624
ZROWS = 208


def _sc_body(x_hbm, src_hbm, dst_hbm, agg_hbm,
             idx_s, idx_d, rows, zbuf, acc, sem):
    c = lax.axis_index("c")
    s = lax.axis_index("s")

    def _zfill(i, carry):
        for j in range(D // 16):
            zbuf[i, pl.ds(j * 16, 16)] = jnp.zeros((16,), jnp.float32)
        return carry
    lax.fori_loop(0, ZROWS, _zfill, 0)
    row0 = s * RPT
    for t in range(RPT // ZROWS):
        pltpu.sync_copy(zbuf, acc.at[pl.ds(row0 + t * ZROWS, ZROWS)])

    @pl.when(s == NS - 1)
    def _():
        pltpu.sync_copy(zbuf.at[pl.ds(0, N - NS * RPT)],
                        acc.at[pl.ds(NS * RPT, N - NS * RPT)])
    plsc.subcore_barrier()

    base = (c * NS + s) * EPT

    def _step(k, carry):
        off = pl.multiple_of(base + k * CH, CH)
        pltpu.sync_copy(src_hbm.at[pl.ds(off, CH)], idx_s)
        pltpu.sync_copy(dst_hbm.at[pl.ds(off, CH)], idx_d)
        pltpu.async_copy(x_hbm.at[idx_s], rows, sem).wait()
        pltpu.sync_copy(rows, acc.at[idx_d], add=True)
        return carry
    lax.fori_loop(0, NCHUNK, _step, 0)

    plsc.subcore_barrier()

    pltpu.sync_copy(acc.at[pl.ds(row0, RPT)], agg_hbm.at[c, pl.ds(row0, RPT)])

    @pl.when(s == NS - 1)
    def _():
        pltpu.sync_copy(acc.at[pl.ds(NS * RPT, N - NS * RPT)],
                        agg_hbm.at[c, pl.ds(NS * RPT, N - NS * RPT)])


@jax.jit
def _sc_scatter(x, src, dst):
    mesh = plsc.VectorSubcoreMesh(core_axis_name="c", subcore_axis_name="s")
    return pl.kernel(
        _sc_body,
        out_type=jax.ShapeDtypeStruct((NC, N, D), jnp.float32),
        mesh=mesh,
        scratch_types=[
            pltpu.VMEM((CH,), jnp.int32),
            pltpu.VMEM((CH,), jnp.int32),
            pltpu.VMEM((CH, D), jnp.float32),
            pltpu.VMEM((ZROWS, D), jnp.float32),
            pltpu.MemorySpace.VMEM_SHARED((N, D), jnp.float32),
            pltpu.SemaphoreType.DMA,
        ],
    )(x, src, dst)


BLK = 400
NBLK = N // BLK


def _tc_body(agg_ref, x_ref, batch_ref, wn_ref, wr_ref, b_ref, out_ref):
    i = pl.program_id(0)
    a = agg_ref[0] + agg_ref[1]
    h = jnp.dot(a, wn_ref[...], preferred_element_type=jnp.float32)
    h = h + jnp.dot(x_ref[...], wr_ref[...], preferred_element_type=jnp.float32)
    h = jnp.maximum(h + b_ref[...], 0.0)
    bt = batch_ref[0, 0, :]
    gid = lax.broadcasted_iota(jnp.int32, (G, BLK), 0)
    onehot = jnp.where(gid == bt[None, :], 1.0, 0.0)
    p = jnp.dot(onehot, h, preferred_element_type=jnp.float32)

    @pl.when(i == 0)
    def _():
        out_ref[...] = p

    @pl.when(i > 0)
    def _():
        out_ref[...] += p


@jax.jit
def _tc_combine(agg2, x, batch3, W_nbr, W_root, b2):
    return pl.pallas_call(
        _tc_body,
        grid=(NBLK,),
        in_specs=[
            pl.BlockSpec((NC, BLK, D), lambda i: (0, i, 0)),
            pl.BlockSpec((BLK, D), lambda i: (i, 0)),
            pl.BlockSpec((1, 1, BLK), lambda i: (i, 0, 0)),
            pl.BlockSpec((D, D), lambda i: (0, 0)),
            pl.BlockSpec((D, D), lambda i: (0, 0)),
            pl.BlockSpec((1, D), lambda i: (0, 0)),
        ],
        out_specs=pl.BlockSpec((G, D), lambda i: (0, 0)),
        out_shape=jax.ShapeDtypeStruct((G, D), jnp.float32),
    )(agg2, x, batch3, W_nbr, W_root, b2)


def kernel(x, edge_index, batch, W_nbr, W_root, b):
    src = edge_index[0]
    dst = edge_index[1]
    agg2 = _sc_scatter(x, src, dst)
    batch3 = batch.reshape(NBLK, 1, BLK)
    b2 = b.reshape(1, D)
    return _tc_combine(agg2, x, batch3, W_nbr, W_root, b2)
